# baseline (speedup 1.0000x reference)
import jax
import jax.numpy as jnp
from jax import lax
from jax.experimental import pallas as pl
from jax.experimental.pallas import tpu as pltpu
from jax.experimental.pallas import tpu_sc as plsc

_VOCAB = 128
_HID = 64
_BLK = 800
_NC = 2
_NS = 16
_NW = _NC * _NS


def _gate_table_body(table_ref, w_ref, b_ref, sig_ref):
    t = table_ref[...]
    w = w_ref[...]
    logits = jnp.sum(t * w, axis=1) + b_ref[0, 0]
    sig_ref[...] = jax.nn.sigmoid(logits)[None, :]


def _sc_body(seq_hbm, table_hbm, sig_hbm, h_hbm, soft_hbm,
             idx_v, rows_v, soft_v, sig_v, table_v,
             sem_i0, sem_i1, sem_wh0, sem_wh1, sem_ws0, sem_ws1):
    wid = lax.axis_index("s") * _NC + lax.axis_index("c")
    n_idx = seq_hbm.shape[0]
    per_w = n_idx // _NW
    n_blk = per_w // _BLK
    base0 = wid * per_w

    sem_i = (sem_i0, sem_i1)
    sem_wh = (sem_wh0, sem_wh1)
    sem_ws = (sem_ws0, sem_ws1)

    pltpu.sync_copy(sig_hbm, sig_v)
    pltpu.sync_copy(table_hbm, table_v)
    for q in (0, 1):
        pltpu.async_copy(seq_hbm.at[pl.ds(base0 + q * _BLK, _BLK)],
                         idx_v.at[q], sem_i[q])

    lanes = lax.broadcasted_iota(jnp.int32, (16,), 0)

    def pair_body(j, carry):
        for q in (0, 1):
            b = 2 * j + q
            pltpu.make_async_copy(seq_hbm.at[pl.ds(0, _BLK)],
                                  idx_v.at[q], sem_i[q]).wait()

            @pl.when(j > 0)
            def _():
                pltpu.make_async_copy(
                    rows_v.at[q], h_hbm.at[pl.ds(0, _BLK * _HID)],
                    sem_wh[q]).wait()
                pltpu.make_async_copy(
                    soft_v.at[q], soft_hbm.at[pl.ds(0, _BLK)],
                    sem_ws[q]).wait()

            @plsc.parallel_loop(0, _BLK // 16, unroll=2)
            def _(t):
                iv = idx_v[q, pl.ds(t * 16, 16)]
                soft_v[q, pl.ds(t * 16, 16)] = plsc.load_gather(sig_v, [iv])
                src0 = iv * _HID
                dst0 = (t * 16 + lanes) * _HID
                for c in range(_HID):
                    vals = plsc.load_gather(table_v, [src0 + c])
                    plsc.store_scatter(rows_v.at[q], [dst0 + c], vals)

            nxt = jnp.minimum(base0 + (b + 2) * _BLK, base0 + per_w - _BLK)
            pltpu.async_copy(seq_hbm.at[pl.ds(nxt, _BLK)],
                             idx_v.at[q], sem_i[q])

            out0 = base0 + b * _BLK
            pltpu.async_copy(rows_v.at[q],
                             h_hbm.at[pl.ds(out0 * _HID, _BLK * _HID)],
                             sem_wh[q])
            pltpu.async_copy(soft_v.at[q], soft_hbm.at[pl.ds(out0, _BLK)],
                             sem_ws[q])
        return carry

    lax.fori_loop(0, n_blk // 2, pair_body, 0)

    for q in (0, 1):
        pltpu.make_async_copy(seq_hbm.at[pl.ds(0, _BLK)],
                              idx_v.at[q], sem_i[q]).wait()
        pltpu.make_async_copy(rows_v.at[q], h_hbm.at[pl.ds(0, _BLK * _HID)],
                              sem_wh[q]).wait()
        pltpu.make_async_copy(soft_v.at[q], soft_hbm.at[pl.ds(0, _BLK)],
                              sem_ws[q]).wait()


def kernel(seq, embed_table, gate_w, gate_b):
    B, L = seq.shape
    n = B * L
    seq1d = seq.reshape(n).astype(jnp.int32)

    sig = pl.pallas_call(
        _gate_table_body,
        out_shape=jax.ShapeDtypeStruct((1, _VOCAB), jnp.float32),
    )(embed_table, gate_w, gate_b.reshape(1, 1))
    sig1d = sig.reshape(_VOCAB)

    mesh = plsc.VectorSubcoreMesh(core_axis_name="c", subcore_axis_name="s",
                                  num_cores=_NC, num_subcores=_NS)
    h_flat, soft1d = pl.kernel(
        _sc_body,
        out_type=[
            jax.ShapeDtypeStruct((n * _HID,), jnp.float32),
            jax.ShapeDtypeStruct((n,), jnp.float32),
        ],
        mesh=mesh,
        scratch_types=[
            pltpu.VMEM((2, _BLK), jnp.int32),
            pltpu.VMEM((2, _BLK * _HID), jnp.float32),
            pltpu.VMEM((2, _BLK), jnp.float32),
            pltpu.VMEM((_VOCAB,), jnp.float32),
            pltpu.VMEM((_VOCAB * _HID,), jnp.float32),
        ] + [pltpu.SemaphoreType.DMA] * 6,
        compiler_params=pltpu.CompilerParams(use_tc_tiling_on_sc=False,
                                             needs_layout_passes=False),
    )(seq1d, embed_table.reshape(_VOCAB * _HID), sig1d)

    h = h_flat.reshape(B, L, _HID)
    soft = soft1d.reshape(B, L)
    return (soft, h)


# speedup vs baseline: 2.1733x; 2.1733x over previous
import jax
import jax.numpy as jnp
from jax import lax
from jax.experimental import pallas as pl
from jax.experimental.pallas import tpu as pltpu
from jax.experimental.pallas import tpu_sc as plsc

_VOCAB = 128
_HID = 64
_BLK = 800
_SBLK = 256
_NC = 2
_NS = 16
_NW = _NC * _NS


def _gate_table_body(table_ref, w_ref, b_ref, sig_ref):
    t = table_ref[...]
    w = w_ref[...]
    logits = jnp.sum(t * w, axis=1) + b_ref[0, 0]
    sig_ref[...] = jax.nn.sigmoid(logits)[None, :]


def _sc_body(seq_hbm, table_hbm, sig_hbm, h_hbm, soft_hbm,
             idx_v, rows_v, soft_v, sig_v, table_v,
             sem_i0, sem_i1, sem_g0, sem_g1,
             sem_wh0, sem_wh1, sem_ws0, sem_ws1):
    wid = lax.axis_index("s") * _NC + lax.axis_index("c")
    n_idx = seq_hbm.shape[0]
    per_w = n_idx // _NW
    n_blk = per_w // _BLK
    base0 = wid * per_w

    sem_i = (sem_i0, sem_i1)
    sem_g = (sem_g0, sem_g1)
    sem_wh = (sem_wh0, sem_wh1)
    sem_ws = (sem_ws0, sem_ws1)

    pltpu.sync_copy(sig_hbm, sig_v)
    pltpu.sync_copy(table_hbm, table_v)
    for q in (0, 1):
        pltpu.async_copy(seq_hbm.at[pl.ds(base0 + q * _BLK, _BLK)],
                         idx_v.at[q], sem_i[q])

    def pair_body(j, carry):
        for q in (0, 1):
            b = 2 * j + q
            pltpu.make_async_copy(seq_hbm.at[pl.ds(0, _BLK)],
                                  idx_v.at[q], sem_i[q]).wait()

            @pl.when(j > 0)
            def _():
                pltpu.make_async_copy(
                    rows_v.at[q], h_hbm.at[pl.ds(0, _BLK)], sem_wh[q]).wait()
                pltpu.make_async_copy(
                    soft_v.at[q], soft_hbm.at[pl.ds(0, _BLK)],
                    sem_ws[q]).wait()

            g = pltpu.async_copy(
                table_hbm.at[idx_v.at[q, pl.ds(0, _SBLK)]],
                rows_v.at[q, pl.ds(0, _SBLK)], sem_g[q])

            @plsc.parallel_loop(_SBLK // 16, _BLK // 16, unroll=2)
            def _(t):
                iv = idx_v[q, pl.ds(t * 16, 16)]
                for r in range(16):
                    vr = iv[r]
                    for c in range(_HID // 16):
                        rows_v[q, t * 16 + r, pl.ds(c * 16, 16)] = (
                            table_v[vr, pl.ds(c * 16, 16)])

            for t in range(_BLK // 16):
                iv = idx_v[q, pl.ds(t * 16, 16)]
                soft_v[q, pl.ds(t * 16, 16)] = plsc.load_gather(sig_v, [iv])

            g.wait()

            nxt = jnp.minimum(base0 + (b + 2) * _BLK, base0 + per_w - _BLK)
            pltpu.async_copy(seq_hbm.at[pl.ds(nxt, _BLK)],
                             idx_v.at[q], sem_i[q])

            out0 = base0 + b * _BLK
            pltpu.async_copy(rows_v.at[q], h_hbm.at[pl.ds(out0, _BLK)],
                             sem_wh[q])
            pltpu.async_copy(soft_v.at[q], soft_hbm.at[pl.ds(out0, _BLK)],
                             sem_ws[q])
        return carry

    lax.fori_loop(0, n_blk // 2, pair_body, 0)

    for q in (0, 1):
        pltpu.make_async_copy(seq_hbm.at[pl.ds(0, _BLK)],
                              idx_v.at[q], sem_i[q]).wait()
        pltpu.make_async_copy(rows_v.at[q], h_hbm.at[pl.ds(0, _BLK)],
                              sem_wh[q]).wait()
        pltpu.make_async_copy(soft_v.at[q], soft_hbm.at[pl.ds(0, _BLK)],
                              sem_ws[q]).wait()


def kernel(seq, embed_table, gate_w, gate_b):
    B, L = seq.shape
    n = B * L
    seq1d = seq.reshape(n).astype(jnp.int32)

    sig = pl.pallas_call(
        _gate_table_body,
        out_shape=jax.ShapeDtypeStruct((1, _VOCAB), jnp.float32),
    )(embed_table, gate_w, gate_b.reshape(1, 1))
    sig1d = sig.reshape(_VOCAB)

    mesh = plsc.VectorSubcoreMesh(core_axis_name="c", subcore_axis_name="s",
                                  num_cores=_NC, num_subcores=_NS)
    h2d, soft1d = pl.kernel(
        _sc_body,
        out_type=[
            jax.ShapeDtypeStruct((n, _HID), jnp.float32),
            jax.ShapeDtypeStruct((n,), jnp.float32),
        ],
        mesh=mesh,
        scratch_types=[
            pltpu.VMEM((2, _BLK), jnp.int32),
            pltpu.VMEM((2, _BLK, _HID), jnp.float32),
            pltpu.VMEM((2, _BLK), jnp.float32),
            pltpu.VMEM((_VOCAB,), jnp.float32),
            pltpu.VMEM((_VOCAB, _HID), jnp.float32),
        ] + [pltpu.SemaphoreType.DMA] * 8,
        compiler_params=pltpu.CompilerParams(use_tc_tiling_on_sc=False,
                                             needs_layout_passes=False),
    )(seq1d, embed_table, sig1d)

    h = h2d.reshape(B, L, _HID)
    soft = soft1d.reshape(B, L)
    return (soft, h)


# speedup vs baseline: 2.3366x; 1.0752x over previous
import jax
import jax.numpy as jnp
from jax import lax
from jax.experimental import pallas as pl
from jax.experimental.pallas import tpu as pltpu
from jax.experimental.pallas import tpu_sc as plsc

_VOCAB = 128
_HID = 64
_BLK = 400
_SBLK = 128
_NC = 2
_NS = 16
_NW = _NC * _NS


def _gate_table_body(table_ref, w_ref, b_ref, sig_ref):
    t = table_ref[...]
    w = w_ref[...]
    logits = jnp.sum(t * w, axis=1) + b_ref[0, 0]
    sig_ref[...] = jax.nn.sigmoid(logits)[None, :]


def _sc_body(seq_hbm, table_hbm, sig_hbm, h_hbm, soft_hbm,
             idx_v, rows_v, soft_v, sig_v, table_v, spm,
             sem_i0, sem_i1, sem_g0, sem_g1, sem_sp0, sem_sp1,
             sem_h0, sem_h1, sem_ws0, sem_ws1):
    cid = lax.axis_index("c")
    sid = lax.axis_index("s")
    wid = sid * _NC + cid
    n_idx = seq_hbm.shape[0]
    per_w = n_idx // _NW
    n_blk = per_w // _BLK
    base0 = wid * per_w

    sem_i = (sem_i0, sem_i1)
    sem_g = (sem_g0, sem_g1)
    sem_sp = (sem_sp0, sem_sp1)
    sem_h = (sem_h0, sem_h1)
    sem_ws = (sem_ws0, sem_ws1)

    pltpu.sync_copy(sig_hbm, sig_v)
    pltpu.sync_copy(table_hbm, table_v)
    for q in (0, 1):
        pltpu.async_copy(seq_hbm.at[pl.ds(base0 + q * _BLK, _BLK)],
                         idx_v.at[q], sem_i[q])

    def pair_body(j, carry):
        for q in (0, 1):
            b = 2 * j + q
            pltpu.make_async_copy(seq_hbm.at[pl.ds(0, _BLK)],
                                  idx_v.at[q], sem_i[q]).wait()

            @pl.when(j > 0)
            def _():
                pltpu.make_async_copy(
                    spm.at[q, sid], h_hbm.at[pl.ds(0, _BLK)],
                    sem_h[q]).wait()
                pltpu.make_async_copy(
                    soft_v.at[q], soft_hbm.at[pl.ds(0, _BLK)],
                    sem_ws[q]).wait()

            @pl.when(b > 0)
            def _():
                pltpu.make_async_copy(
                    rows_v.at[1 - q], spm.at[1 - q, sid],
                    sem_sp[1 - q]).wait()
                pltpu.async_copy(
                    spm.at[1 - q, sid],
                    h_hbm.at[pl.ds(base0 + (b - 1) * _BLK, _BLK)],
                    sem_h[1 - q])

            g = pltpu.async_copy(
                table_hbm.at[idx_v.at[q, pl.ds(0, _SBLK)]],
                rows_v.at[q, pl.ds(0, _SBLK)], sem_g[q])

            @plsc.parallel_loop(_SBLK // 16, _BLK // 16, unroll=2)
            def _(t):
                iv = idx_v[q, pl.ds(t * 16, 16)]
                for r in range(16):
                    vr = iv[r]
                    for c in range(_HID // 16):
                        rows_v[q, t * 16 + r, pl.ds(c * 16, 16)] = (
                            table_v[vr, pl.ds(c * 16, 16)])

            for t in range(_BLK // 16):
                iv = idx_v[q, pl.ds(t * 16, 16)]
                soft_v[q, pl.ds(t * 16, 16)] = plsc.load_gather(sig_v, [iv])

            g.wait()

            nxt = jnp.minimum(base0 + (b + 2) * _BLK, base0 + per_w - _BLK)
            pltpu.async_copy(seq_hbm.at[pl.ds(nxt, _BLK)],
                             idx_v.at[q], sem_i[q])

            pltpu.async_copy(rows_v.at[q], spm.at[q, sid], sem_sp[q])
            pltpu.async_copy(soft_v.at[q],
                             soft_hbm.at[pl.ds(base0 + b * _BLK, _BLK)],
                             sem_ws[q])
        return carry

    lax.fori_loop(0, n_blk // 2, pair_body, 0)

    last = n_blk - 1
    pltpu.make_async_copy(rows_v.at[1], spm.at[1, sid], sem_sp[1]).wait()
    pltpu.async_copy(spm.at[1, sid],
                     h_hbm.at[pl.ds(base0 + last * _BLK, _BLK)], sem_h[1])
    for q in (0, 1):
        pltpu.make_async_copy(seq_hbm.at[pl.ds(0, _BLK)],
                              idx_v.at[q], sem_i[q]).wait()
        pltpu.make_async_copy(spm.at[q, sid], h_hbm.at[pl.ds(0, _BLK)],
                              sem_h[q]).wait()
        pltpu.make_async_copy(soft_v.at[q], soft_hbm.at[pl.ds(0, _BLK)],
                              sem_ws[q]).wait()


def kernel(seq, embed_table, gate_w, gate_b):
    B, L = seq.shape
    n = B * L
    seq1d = seq.reshape(n).astype(jnp.int32)

    sig = pl.pallas_call(
        _gate_table_body,
        out_shape=jax.ShapeDtypeStruct((1, _VOCAB), jnp.float32),
    )(embed_table, gate_w, gate_b.reshape(1, 1))
    sig1d = sig.reshape(_VOCAB)

    mesh = plsc.VectorSubcoreMesh(core_axis_name="c", subcore_axis_name="s",
                                  num_cores=_NC, num_subcores=_NS)
    h2d, soft1d = pl.kernel(
        _sc_body,
        out_type=[
            jax.ShapeDtypeStruct((n, _HID), jnp.float32),
            jax.ShapeDtypeStruct((n,), jnp.float32),
        ],
        mesh=mesh,
        scratch_types=[
            pltpu.VMEM((2, _BLK), jnp.int32),
            pltpu.VMEM((2, _BLK, _HID), jnp.float32),
            pltpu.VMEM((2, _BLK), jnp.float32),
            pltpu.VMEM((_VOCAB,), jnp.float32),
            pltpu.VMEM((_VOCAB, _HID), jnp.float32),
            pltpu.VMEM_SHARED((2, _NS, _BLK, _HID), jnp.float32),
        ] + [pltpu.SemaphoreType.DMA] * 10,
        compiler_params=pltpu.CompilerParams(use_tc_tiling_on_sc=False,
                                             needs_layout_passes=False),
    )(seq1d, embed_table, sig1d)

    h = h2d.reshape(B, L, _HID)
    soft = soft1d.reshape(B, L)
    return (soft, h)


# speedup vs baseline: 3.7127x; 1.5889x over previous
import jax
import jax.numpy as jnp
from jax import lax
from jax.experimental import pallas as pl
from jax.experimental.pallas import tpu as pltpu
from jax.experimental.pallas import tpu_sc as plsc

_VOCAB = 128
_HID = 64
_BLK = 6400
_TBLK = 2048
_NC = 2
_NS = 16
_NW = _NC * _NS


def _gate_table_body(table_ref, w_ref, b_ref, sig_ref):
    t = table_ref[...]
    w = w_ref[...]
    logits = jnp.sum(t * w, axis=1) + b_ref[0, 0]
    sig_ref[...] = jax.nn.sigmoid(logits)[None, :]


def _h_expand_body(seq_ref, table_ref, h_ref):
    idx = seq_ref[0, 0, :]
    oh = (idx[:, None] == lax.broadcasted_iota(
        jnp.int32, (_TBLK, _VOCAB), 1)).astype(jnp.float32)
    h_ref[...] = jnp.dot(oh, table_ref[...],
                         preferred_element_type=jnp.float32)


def _sc_soft_body(seq_hbm, sig_hbm, soft_hbm,
                  idx_v, soft_v, sig_v,
                  sem_i0, sem_i1, sem_ws0, sem_ws1):
    wid = lax.axis_index("s") * _NC + lax.axis_index("c")
    n_idx = seq_hbm.shape[0]
    per_w = n_idx // _NW
    n_blk = per_w // _BLK
    base0 = wid * per_w

    sem_i = (sem_i0, sem_i1)
    sem_ws = (sem_ws0, sem_ws1)

    pltpu.sync_copy(sig_hbm, sig_v)
    for q in (0, 1):
        pltpu.async_copy(seq_hbm.at[pl.ds(base0 + q * _BLK, _BLK)],
                         idx_v.at[q], sem_i[q])

    def pair_body(j, carry):
        for q in (0, 1):
            b = 2 * j + q
            pltpu.make_async_copy(seq_hbm.at[pl.ds(0, _BLK)],
                                  idx_v.at[q], sem_i[q]).wait()

            @pl.when(j > 0)
            def _():
                pltpu.make_async_copy(
                    soft_v.at[q], soft_hbm.at[pl.ds(0, _BLK)],
                    sem_ws[q]).wait()

            @plsc.parallel_loop(0, _BLK // 16, unroll=4)
            def _(t):
                iv = idx_v[q, pl.ds(t * 16, 16)]
                soft_v[q, pl.ds(t * 16, 16)] = plsc.load_gather(sig_v, [iv])

            nxt = jnp.minimum(base0 + (b + 2) * _BLK, base0 + per_w - _BLK)
            pltpu.async_copy(seq_hbm.at[pl.ds(nxt, _BLK)],
                             idx_v.at[q], sem_i[q])
            pltpu.async_copy(soft_v.at[q],
                             soft_hbm.at[pl.ds(base0 + b * _BLK, _BLK)],
                             sem_ws[q])
        return carry

    lax.fori_loop(0, n_blk // 2, pair_body, 0)

    for q in (0, 1):
        pltpu.make_async_copy(seq_hbm.at[pl.ds(0, _BLK)],
                              idx_v.at[q], sem_i[q]).wait()
        pltpu.make_async_copy(soft_v.at[q], soft_hbm.at[pl.ds(0, _BLK)],
                              sem_ws[q]).wait()


def kernel(seq, embed_table, gate_w, gate_b):
    B, L = seq.shape
    n = B * L
    seq1d = seq.reshape(n).astype(jnp.int32)

    sig = pl.pallas_call(
        _gate_table_body,
        out_shape=jax.ShapeDtypeStruct((1, _VOCAB), jnp.float32),
    )(embed_table, gate_w, gate_b.reshape(1, 1))
    sig1d = sig.reshape(_VOCAB)

    mesh = plsc.VectorSubcoreMesh(core_axis_name="c", subcore_axis_name="s",
                                  num_cores=_NC, num_subcores=_NS)
    soft1d = pl.kernel(
        _sc_soft_body,
        out_type=jax.ShapeDtypeStruct((n,), jnp.float32),
        mesh=mesh,
        scratch_types=[
            pltpu.VMEM((2, _BLK), jnp.int32),
            pltpu.VMEM((2, _BLK), jnp.float32),
            pltpu.VMEM((_VOCAB,), jnp.float32),
        ] + [pltpu.SemaphoreType.DMA] * 4,
        compiler_params=pltpu.CompilerParams(use_tc_tiling_on_sc=False,
                                             needs_layout_passes=False),
    )(seq1d, sig1d)

    seq3d = seq1d.reshape(n // _TBLK, 1, _TBLK)
    h2d = pl.pallas_call(
        _h_expand_body,
        grid=(n // _TBLK,),
        in_specs=[
            pl.BlockSpec((1, 1, _TBLK), lambda i: (i, 0, 0)),
            pl.BlockSpec((_VOCAB, _HID), lambda i: (0, 0)),
        ],
        out_specs=pl.BlockSpec((_TBLK, _HID), lambda i: (i, 0)),
        out_shape=jax.ShapeDtypeStruct((n, _HID), jnp.float32),
    )(seq3d, embed_table)

    h = h2d.reshape(B, L, _HID)
    soft = soft1d.reshape(B, L)
    return (soft, h)


# speedup vs baseline: 4.0572x; 1.0928x over previous
import jax
import jax.numpy as jnp
from jax import lax
from jax.experimental import pallas as pl
from jax.experimental.pallas import tpu as pltpu
from jax.experimental.pallas import tpu_sc as plsc

_VOCAB = 128
_HID = 64
_BLK = 6400
_TB = 32
_NC = 2
_NS = 16
_NW = _NC * _NS


def _gate_table_body(table_ref, w_ref, b_ref, sig_ref):
    t = table_ref[...]
    w = w_ref[...]
    logits = jnp.sum(t * w, axis=1) + b_ref[0, 0]
    sig_ref[...] = jax.nn.sigmoid(logits)[None, :]


def _h_expand_body(seq_ref, table_ref, h_ref):
    idx = seq_ref[...]
    oh = (idx[:, :, None] == lax.broadcasted_iota(
        jnp.int32, idx.shape + (_VOCAB,), 2)).astype(jnp.float32)
    h_ref[...] = lax.dot_general(
        oh, table_ref[...], (((2,), (0,)), ((), ())),
        preferred_element_type=jnp.float32)


def _sc_soft_body(seq_hbm, sig_hbm, soft_hbm,
                  idx_v, soft_v, sig_v,
                  sem_i0, sem_i1, sem_ws0, sem_ws1):
    wid = lax.axis_index("s") * _NC + lax.axis_index("c")
    n_idx = seq_hbm.shape[0]
    per_w = n_idx // _NW
    n_blk = per_w // _BLK
    base0 = wid * per_w

    sem_i = (sem_i0, sem_i1)
    sem_ws = (sem_ws0, sem_ws1)

    pltpu.sync_copy(sig_hbm, sig_v)
    for q in (0, 1):
        pltpu.async_copy(seq_hbm.at[pl.ds(base0 + q * _BLK, _BLK)],
                         idx_v.at[q], sem_i[q])

    def pair_body(j, carry):
        for q in (0, 1):
            b = 2 * j + q
            pltpu.make_async_copy(seq_hbm.at[pl.ds(0, _BLK)],
                                  idx_v.at[q], sem_i[q]).wait()

            @pl.when(j > 0)
            def _():
                pltpu.make_async_copy(
                    soft_v.at[q], soft_hbm.at[pl.ds(0, _BLK)],
                    sem_ws[q]).wait()

            @plsc.parallel_loop(0, _BLK // 16, unroll=4)
            def _(t):
                iv = idx_v[q, pl.ds(t * 16, 16)]
                soft_v[q, pl.ds(t * 16, 16)] = plsc.load_gather(sig_v, [iv])

            nxt = jnp.minimum(base0 + (b + 2) * _BLK, base0 + per_w - _BLK)
            pltpu.async_copy(seq_hbm.at[pl.ds(nxt, _BLK)],
                             idx_v.at[q], sem_i[q])
            pltpu.async_copy(soft_v.at[q],
                             soft_hbm.at[pl.ds(base0 + b * _BLK, _BLK)],
                             sem_ws[q])
        return carry

    lax.fori_loop(0, n_blk // 2, pair_body, 0)

    for q in (0, 1):
        pltpu.make_async_copy(seq_hbm.at[pl.ds(0, _BLK)],
                              idx_v.at[q], sem_i[q]).wait()
        pltpu.make_async_copy(soft_v.at[q], soft_hbm.at[pl.ds(0, _BLK)],
                              sem_ws[q]).wait()


def kernel(seq, embed_table, gate_w, gate_b):
    B, L = seq.shape
    n = B * L
    seq1d = seq.reshape(n).astype(jnp.int32)

    sig = pl.pallas_call(
        _gate_table_body,
        out_shape=jax.ShapeDtypeStruct((1, _VOCAB), jnp.float32),
    )(embed_table, gate_w, gate_b.reshape(1, 1))
    sig1d = sig.reshape(_VOCAB)

    mesh = plsc.VectorSubcoreMesh(core_axis_name="c", subcore_axis_name="s",
                                  num_cores=_NC, num_subcores=_NS)
    soft1d = pl.kernel(
        _sc_soft_body,
        out_type=jax.ShapeDtypeStruct((n,), jnp.float32),
        mesh=mesh,
        scratch_types=[
            pltpu.VMEM((2, _BLK), jnp.int32),
            pltpu.VMEM((2, _BLK), jnp.float32),
            pltpu.VMEM((_VOCAB,), jnp.float32),
        ] + [pltpu.SemaphoreType.DMA] * 4,
        compiler_params=pltpu.CompilerParams(use_tc_tiling_on_sc=False,
                                             needs_layout_passes=False),
    )(seq1d, sig1d)

    h = pl.pallas_call(
        _h_expand_body,
        grid=(B // _TB,),
        in_specs=[
            pl.BlockSpec((_TB, L), lambda i: (i, 0)),
            pl.BlockSpec((_VOCAB, _HID), lambda i: (0, 0)),
        ],
        out_specs=pl.BlockSpec((_TB, L, _HID), lambda i: (i, 0, 0)),
        out_shape=jax.ShapeDtypeStruct((B, L, _HID), jnp.float32),
    )(seq.astype(jnp.int32), embed_table)

    soft = soft1d.reshape(B, L)
    return (soft, h)


# speedup vs baseline: 4.3182x; 1.0643x over previous
import jax
import jax.numpy as jnp
from jax import lax
from jax.experimental import pallas as pl
from jax.experimental.pallas import tpu as pltpu
from jax.experimental.pallas import tpu_sc as plsc

_VOCAB = 128
_HID = 64
_BLK = 6400
_TB = 64
_NC = 2
_NS = 16
_NW = _NC * _NS


def _gate_table_body(table_ref, w_ref, b_ref, sig_ref):
    t = table_ref[...]
    w = w_ref[...]
    logits = jnp.sum(t * w, axis=1) + b_ref[0, 0]
    sig_ref[...] = jax.nn.sigmoid(logits)[None, :]


def _h_expand_body(seq_ref, table_ref, h_ref):
    idx = seq_ref[...]
    oh = (idx[:, :, None] == lax.broadcasted_iota(
        jnp.int32, idx.shape + (_VOCAB,), 2)).astype(jnp.float32)
    h_ref[...] = lax.dot_general(
        oh, table_ref[...], (((2,), (0,)), ((), ())),
        preferred_element_type=jnp.float32)


def _sc_soft_body(seq_hbm, sig_hbm, soft_hbm,
                  idx_v, soft_v, sig_v,
                  sem_i0, sem_i1, sem_ws0, sem_ws1):
    wid = lax.axis_index("s") * _NC + lax.axis_index("c")
    n_idx = seq_hbm.shape[0]
    per_w = n_idx // _NW
    n_blk = per_w // _BLK
    base0 = wid * per_w

    sem_i = (sem_i0, sem_i1)
    sem_ws = (sem_ws0, sem_ws1)

    pltpu.sync_copy(sig_hbm, sig_v)
    for q in (0, 1):
        pltpu.async_copy(seq_hbm.at[pl.ds(base0 + q * _BLK, _BLK)],
                         idx_v.at[q], sem_i[q])

    def pair_body(j, carry):
        for q in (0, 1):
            b = 2 * j + q
            pltpu.make_async_copy(seq_hbm.at[pl.ds(0, _BLK)],
                                  idx_v.at[q], sem_i[q]).wait()

            @pl.when(j > 0)
            def _():
                pltpu.make_async_copy(
                    soft_v.at[q], soft_hbm.at[pl.ds(0, _BLK)],
                    sem_ws[q]).wait()

            @plsc.parallel_loop(0, _BLK // 16, unroll=4)
            def _(t):
                iv = idx_v[q, pl.ds(t * 16, 16)]
                soft_v[q, pl.ds(t * 16, 16)] = plsc.load_gather(sig_v, [iv])

            nxt = jnp.minimum(base0 + (b + 2) * _BLK, base0 + per_w - _BLK)
            pltpu.async_copy(seq_hbm.at[pl.ds(nxt, _BLK)],
                             idx_v.at[q], sem_i[q])
            pltpu.async_copy(soft_v.at[q],
                             soft_hbm.at[pl.ds(base0 + b * _BLK, _BLK)],
                             sem_ws[q])
        return carry

    lax.fori_loop(0, n_blk // 2, pair_body, 0)

    for q in (0, 1):
        pltpu.make_async_copy(seq_hbm.at[pl.ds(0, _BLK)],
                              idx_v.at[q], sem_i[q]).wait()
        pltpu.make_async_copy(soft_v.at[q], soft_hbm.at[pl.ds(0, _BLK)],
                              sem_ws[q]).wait()


def kernel(seq, embed_table, gate_w, gate_b):
    B, L = seq.shape
    n = B * L
    seq1d = seq.reshape(n).astype(jnp.int32)

    sig = pl.pallas_call(
        _gate_table_body,
        out_shape=jax.ShapeDtypeStruct((1, _VOCAB), jnp.float32),
    )(embed_table, gate_w, gate_b.reshape(1, 1))
    sig1d = sig.reshape(_VOCAB)

    mesh = plsc.VectorSubcoreMesh(core_axis_name="c", subcore_axis_name="s",
                                  num_cores=_NC, num_subcores=_NS)
    soft1d = pl.kernel(
        _sc_soft_body,
        out_type=jax.ShapeDtypeStruct((n,), jnp.float32),
        mesh=mesh,
        scratch_types=[
            pltpu.VMEM((2, _BLK), jnp.int32),
            pltpu.VMEM((2, _BLK), jnp.float32),
            pltpu.VMEM((_VOCAB,), jnp.float32),
        ] + [pltpu.SemaphoreType.DMA] * 4,
        compiler_params=pltpu.CompilerParams(use_tc_tiling_on_sc=False,
                                             needs_layout_passes=False),
    )(seq1d, sig1d)

    h = pl.pallas_call(
        _h_expand_body,
        grid=(B // _TB,),
        in_specs=[
            pl.BlockSpec((_TB, L), lambda i: (i, 0)),
            pl.BlockSpec((_VOCAB, _HID), lambda i: (0, 0)),
        ],
        out_specs=pl.BlockSpec((_TB, L, _HID), lambda i: (i, 0, 0)),
        out_shape=jax.ShapeDtypeStruct((B, L, _HID), jnp.float32),
    )(seq.astype(jnp.int32), embed_table)

    soft = soft1d.reshape(B, L)
    return (soft, h)
